# baseline (speedup 1.0000x reference)
import dataclasses
import functools

import jax
import jax.numpy as jnp
from jax import lax
from jax.experimental import pallas as pl
from jax.experimental.pallas import tpu as pltpu
from jax.experimental.pallas import tpu_sc as plsc

B, S, H = 1, 2048, 1024
E, K, I = 8, 2, 512
T = B * S
TILE = 256
NTR = 24
ROWS = NTR * TILE
SH_TILE = 512

NC, NS = 2, 16
NW = NC * NS


def _router_kernel(x_ref, wg_ref, pw_ref, pos_ref, te_ref):
    logits = jnp.dot(x_ref[...], wg_ref[...].T,
                     preferred_element_type=jnp.float32)
    m1 = jnp.max(logits, axis=-1, keepdims=True)
    sel1 = (logits == m1).astype(jnp.float32)
    masked = jnp.where(sel1 > 0, -jnp.inf, logits)
    m2 = jnp.max(masked, axis=-1, keepdims=True)
    sel2 = (masked == m2).astype(jnp.float32)

    e2 = jnp.exp(m2 - m1)
    denom = 1.0 + e2 + 1e-20
    w1 = 1.0 / denom
    w2 = e2 / denom
    col = lax.broadcasted_iota(jnp.int32, (T, E), 1)
    pw_ref[...] = jnp.where(col == 0, w1, jnp.where(col == 1, w2, 0.0))

    def _cumsum0(a):
        c = a
        k = 1
        while k < T:
            c = c + jnp.concatenate(
                [jnp.zeros((k, E), a.dtype), c[:-k]], axis=0)
            k *= 2
        return c

    c1 = _cumsum0(sel1)
    c2 = _cumsum0(sel2)
    rank1 = c1 - sel1
    rank2 = c2 - sel2
    count1 = c1[-1:, :]
    count2 = c2[-1:, :]
    count = count1 + count2
    tiles = jnp.floor((count + (TILE - 1)) * (1.0 / TILE))
    lo = lax.broadcasted_iota(jnp.int32, (E, E), 0)
    hi = lax.broadcasted_iota(jnp.int32, (E, E), 1)
    cumt = jnp.dot(tiles, (lo <= hi).astype(jnp.float32),
                   preferred_element_type=jnp.float32)
    row_start = TILE * (cumt - tiles)

    p1 = jnp.sum(sel1 * (row_start + rank1), axis=1, keepdims=True)
    p2 = jnp.sum(sel2 * (row_start + count1 + rank2), axis=1, keepdims=True)
    pos_ref[...] = jnp.where(col == 0, p1.astype(jnp.int32),
                             jnp.where(col == 1, p2.astype(jnp.int32), 0))

    ti = lax.broadcasted_iota(jnp.int32, (NTR, E), 0).astype(jnp.float32)
    te = jnp.sum((ti >= cumt).astype(jnp.int32), axis=1, keepdims=True)
    te_ref[...] = jnp.broadcast_to(te, (NTR, E))


def _scatter_kernel(pos_hbm, src_hbm, pos_v, src_v):
    wid = lax.axis_index("s") * NC + lax.axis_index("c")

    @pl.when(wid == 0)
    def _():
        pltpu.sync_copy(pos_hbm, pos_v)
        iota16 = lax.iota(jnp.int32, 16)

        @pl.loop(0, ROWS, step=16)
        def _(i):
            src_v[pl.ds(i, 16)] = (iota16 + i) & (T - 1)

        @pl.loop(0, T, step=16)
        def _(i):
            tok = iota16 + i
            p1 = plsc.load_gather(pos_v, [tok * E])
            plsc.store_scatter(src_v, [p1], tok)
            p2 = plsc.load_gather(pos_v, [tok * E + 1])
            plsc.store_scatter(src_v, [p2], tok)

        pltpu.sync_copy(src_v, src_hbm)


def _gather_kernel(table_hbm, idx_hbm, out_hbm, idx_v, rows_v, sem):
    wid = lax.axis_index("s") * NC + lax.axis_index("c")
    per_w = ROWS // NW
    chunk = 96
    base = wid * per_w
    pltpu.sync_copy(idx_hbm.at[pl.ds(base, per_w)], idx_v)
    for k in range(per_w // chunk):
        pltpu.async_copy(table_hbm.at[idx_v.at[pl.ds(k * chunk, chunk)]],
                         rows_v, sem).wait()
        pltpu.sync_copy(rows_v, out_hbm.at[pl.ds(base + k * chunk, chunk)])


def _ffn_body(x_ref, wg_ref, wu_ref, wd_ref, o_ref):
    xb = x_ref[...].astype(jnp.bfloat16)
    g = jnp.dot(xb, wg_ref[0].T, preferred_element_type=jnp.float32)
    u = jnp.dot(xb, wu_ref[0].T, preferred_element_type=jnp.float32)
    hmid = (g * jax.nn.sigmoid(g)) * u
    o_ref[...] = jnp.dot(hmid.astype(jnp.bfloat16), wd_ref[0].T,
                         preferred_element_type=jnp.float32)


def _ffn_routed_kernel(pf_ref, x_ref, wg_ref, wu_ref, wd_ref, o_ref):
    @pl.when(pf_ref[pl.program_id(0)] < E)
    def _():
        _ffn_body(x_ref, wg_ref, wu_ref, wd_ref, o_ref)


def _combine_kernel(pw_ref, r1_ref, r2_ref, sh_ref, o_ref):
    col = lax.broadcasted_iota(jnp.int32, (SH_TILE, E), 1)
    pw = pw_ref[...]
    w1 = jnp.sum(jnp.where(col == 0, pw, 0.0), axis=1, keepdims=True)
    w2 = jnp.sum(jnp.where(col == 1, pw, 0.0), axis=1, keepdims=True)
    o_ref[...] = w1 * r1_ref[...] + w2 * r2_ref[...] + sh_ref[...]


@jax.jit
def kernel(x, Wg, We_gate, We_up, We_down, Ws_gate, Ws_up, Ws_down):
    xf = x.reshape(T, H)

    pw, pos, te_mat = pl.pallas_call(
        _router_kernel,
        out_shape=(
            jax.ShapeDtypeStruct((T, E), jnp.float32),
            jax.ShapeDtypeStruct((T, E), jnp.int32),
            jax.ShapeDtypeStruct((NTR, E), jnp.int32),
        ),
    )(xf, Wg)

    sh = pl.pallas_call(
        _ffn_body,
        grid=(T // SH_TILE,),
        in_specs=[
            pl.BlockSpec((SH_TILE, H), lambda i: (i, 0)),
            pl.BlockSpec((1, I, H), lambda i: (0, 0, 0)),
            pl.BlockSpec((1, I, H), lambda i: (0, 0, 0)),
            pl.BlockSpec((1, H, I), lambda i: (0, 0, 0)),
        ],
        out_specs=pl.BlockSpec((SH_TILE, H), lambda i: (i, 0)),
        out_shape=jax.ShapeDtypeStruct((T, H), jnp.float32),
    )(xf, Ws_gate.astype(jnp.bfloat16)[None],
      Ws_up.astype(jnp.bfloat16)[None], Ws_down.astype(jnp.bfloat16)[None])

    mesh = plsc.VectorSubcoreMesh(core_axis_name="c", subcore_axis_name="s")
    sc_params = pltpu.CompilerParams()
    if "needs_layout_passes" in pltpu.CompilerParams.__dataclass_fields__:
        sc_params = dataclasses.replace(sc_params, needs_layout_passes=False)

    src = pl.kernel(
        _scatter_kernel,
        out_type=jax.ShapeDtypeStruct((ROWS,), jnp.int32),
        mesh=mesh,
        scratch_types=[pltpu.VMEM((T * E,), jnp.int32),
                       pltpu.VMEM((ROWS,), jnp.int32)],
        compiler_params=sc_params,
    )(pos.reshape(-1))

    x_sorted = pl.kernel(
        _gather_kernel,
        out_type=jax.ShapeDtypeStruct((ROWS, H), jnp.float32),
        mesh=mesh,
        scratch_types=[pltpu.VMEM((ROWS // NW,), jnp.int32),
                       pltpu.VMEM((96, H), jnp.float32),
                       pltpu.SemaphoreType.DMA],
        compiler_params=sc_params,
    )(xf, src)

    te = te_mat[:, 0]
    pf = jnp.minimum(te, E).astype(jnp.int32)

    we_g = We_gate.astype(jnp.bfloat16)
    we_u = We_up.astype(jnp.bfloat16)
    we_d = We_down.astype(jnp.bfloat16)

    os_ = pl.pallas_call(
        _ffn_routed_kernel,
        grid_spec=pltpu.PrefetchScalarGridSpec(
            num_scalar_prefetch=1,
            grid=(NTR,),
            in_specs=[
                pl.BlockSpec((TILE, H), lambda i, pf: (i, 0)),
                pl.BlockSpec((1, I, H),
                             lambda i, pf: (jnp.minimum(pf[i], E - 1), 0, 0)),
                pl.BlockSpec((1, I, H),
                             lambda i, pf: (jnp.minimum(pf[i], E - 1), 0, 0)),
                pl.BlockSpec((1, H, I),
                             lambda i, pf: (jnp.minimum(pf[i], E - 1), 0, 0)),
            ],
            out_specs=pl.BlockSpec((TILE, H), lambda i, pf: (i, 0)),
        ),
        out_shape=jax.ShapeDtypeStruct((ROWS, H), jnp.float32),
        compiler_params=pltpu.CompilerParams(
            dimension_semantics=("arbitrary",),
        ),
    )(pf, x_sorted, we_g, we_u, we_d)

    pos1 = pos[:, 0]
    pos2 = pos[:, 1]

    def _pair_gather(os_hbm, i1_hbm, i2_hbm, r1_hbm, r2_hbm,
                     idx_v, rows_v, sem):
        wid = lax.axis_index("s") * NC + lax.axis_index("c")
        base = wid * (T // NW)
        pltpu.sync_copy(i1_hbm.at[pl.ds(base, T // NW)], idx_v)
        pltpu.async_copy(os_hbm.at[idx_v], rows_v, sem).wait()
        pltpu.sync_copy(rows_v, r1_hbm.at[pl.ds(base, T // NW)])
        pltpu.sync_copy(i2_hbm.at[pl.ds(base, T // NW)], idx_v)
        pltpu.async_copy(os_hbm.at[idx_v], rows_v, sem).wait()
        pltpu.sync_copy(rows_v, r2_hbm.at[pl.ds(base, T // NW)])

    r1, r2 = pl.kernel(
        _pair_gather,
        out_type=(jax.ShapeDtypeStruct((T, H), jnp.float32),
                  jax.ShapeDtypeStruct((T, H), jnp.float32)),
        mesh=mesh,
        scratch_types=[pltpu.VMEM((T // NW,), jnp.int32),
                       pltpu.VMEM((T // NW, H), jnp.float32),
                       pltpu.SemaphoreType.DMA],
        compiler_params=sc_params,
    )(os_, pos1, pos2)

    y = pl.pallas_call(
        _combine_kernel,
        grid=(T // SH_TILE,),
        in_specs=[
            pl.BlockSpec((SH_TILE, E), lambda i: (i, 0)),
            pl.BlockSpec((SH_TILE, H), lambda i: (i, 0)),
            pl.BlockSpec((SH_TILE, H), lambda i: (i, 0)),
            pl.BlockSpec((SH_TILE, H), lambda i: (i, 0)),
        ],
        out_specs=pl.BlockSpec((SH_TILE, H), lambda i: (i, 0)),
        out_shape=jax.ShapeDtypeStruct((T, H), jnp.float32),
    )(pw, r1, r2, sh)
    return y.reshape(B, S, H)
